# baseline (speedup 1.0000x reference)
import jax
import jax.numpy as jnp
from jax.experimental import pallas as pl
from jax.experimental.pallas import tpu as pltpu

_D = 512
_P = 8
_T = 128
_BB = 32


def _body(xt_ref, w_ref, pos_ref, o_ref):
    w = w_ref[...]
    add = pos_ref[...]
    for b in range(_BB):
        xb = xt_ref[b]
        yb = jax.lax.dot_general(
            xb, w,
            (((0,), (0,)), ((), ())),
            preferred_element_type=jnp.float32,
        )
        o_ref[b] = yb + add


def kernel(inputs, W_proj, b_proj, pos_table):
    B = inputs.shape[0]
    xt = jnp.swapaxes(inputs.reshape(B, _T, _P), 1, 2)
    posb = pos_table[:_T] + b_proj
    return pl.pallas_call(
        _body,
        grid=(B // _BB,),
        in_specs=[
            pl.BlockSpec((_BB, _P, _T), lambda i: (i, 0, 0)),
            pl.BlockSpec((_P, _D), lambda i: (0, 0)),
            pl.BlockSpec((_T, _D), lambda i: (0, 0)),
        ],
        out_specs=pl.BlockSpec((_BB, _T, _D), lambda i: (i, 0, 0)),
        out_shape=jax.ShapeDtypeStruct((B, _T, _D), jnp.float32),
        compiler_params=pltpu.CompilerParams(
            dimension_semantics=("parallel",),
        ),
    )(xt, W_proj, posb)


# speedup vs baseline: 1.0236x; 1.0236x over previous
import jax
import jax.numpy as jnp
from jax.experimental import pallas as pl
from jax.experimental.pallas import tpu as pltpu

_D = 512
_P = 8
_T = 128
_BB = 32


def _body(xt_ref, w_ref, b_ref, pos_ref, o_ref):
    w = w_ref[...]
    add = pos_ref[...] + b_ref[...]
    for b in range(_BB):
        xb = xt_ref[b]
        yb = jax.lax.dot_general(
            xb, w,
            (((0,), (0,)), ((), ())),
            preferred_element_type=jnp.float32,
        )
        o_ref[b] = yb + add


def kernel(inputs, W_proj, b_proj, pos_table):
    B = inputs.shape[0]
    xt = jnp.swapaxes(inputs.reshape(B, _T, _P), 1, 2)
    b2 = b_proj.reshape(1, _D)
    return pl.pallas_call(
        _body,
        grid=(B // _BB,),
        in_specs=[
            pl.BlockSpec((_BB, _P, _T), lambda i: (i, 0, 0)),
            pl.BlockSpec((_P, _D), lambda i: (0, 0)),
            pl.BlockSpec((1, _D), lambda i: (0, 0)),
            pl.BlockSpec((_T, _D), lambda i: (0, 0)),
        ],
        out_specs=pl.BlockSpec((_BB, _T, _D), lambda i: (i, 0, 0)),
        out_shape=jax.ShapeDtypeStruct((B, _T, _D), jnp.float32),
        compiler_params=pltpu.CompilerParams(
            dimension_semantics=("parallel",),
        ),
    )(xt, W_proj, b2, pos_table)
